# baseline (speedup 1.0000x reference)
import functools

import jax
import jax.numpy as jnp
from jax import lax
from jax.experimental import pallas as pl
from jax.experimental.pallas import tpu as pltpu
from jax.experimental.pallas import tpu_sc as plsc

_B = 16384
_D = 64
_CHUNK = 256


def _worker_id():
    info = plsc.get_sparse_core_info()
    return lax.axis_index("s") * info.num_cores + lax.axis_index("c")


def _enqueue_rows(tbl, idx_v, buf, sem, c):
    def body(g, carry):
        vec = idx_v[pl.ds(c * _CHUNK + g * 16, 16)]
        for k in range(16):
            pltpu.async_copy(tbl.at[vec[k]], buf.at[g * 16 + k], sem)
        return carry
    lax.fori_loop(0, _CHUNK // 16, body, 0)


def _mesh():
    return plsc.VectorSubcoreMesh(core_axis_name="c", subcore_axis_name="s")


def kernel(user_table, item_table, user_idx, item_idx):
    info = plsc.get_sparse_core_info()
    nw = info.num_cores * info.num_subcores
    bpw = _B // nw
    nchunk = bpw // _CHUNK

    @functools.partial(
        pl.kernel,
        mesh=_mesh(),
        out_type=jax.ShapeDtypeStruct((_B, _D), jnp.float32),
        scratch_types=[
            pltpu.VMEM((bpw,), jnp.int32),
            pltpu.VMEM((_CHUNK, _D), jnp.float32),
            pltpu.VMEM((_CHUNK, _D), jnp.float32),
            pltpu.SemaphoreType.DMA,
            pltpu.SemaphoreType.DMA,
        ],
        compiler_params=pltpu.CompilerParams(use_tc_tiling_on_sc=True),
    )
    def _gather_user(tbl, idx, out, idx_v, buf_a, buf_b, sem_a, sem_b):
        base = _worker_id() * bpw
        pltpu.sync_copy(idx.at[pl.ds(base, bpw)], idx_v)
        _enqueue_rows(tbl, idx_v, buf_a, sem_a, 0)
        _enqueue_rows(tbl, idx_v, buf_b, sem_b, 1)
        pltpu.make_async_copy(tbl.at[pl.ds(0, _CHUNK)], buf_a, sem_a).wait()
        pltpu.sync_copy(buf_a, out.at[pl.ds(base, _CHUNK)])
        pltpu.make_async_copy(tbl.at[pl.ds(0, _CHUNK)], buf_b, sem_b).wait()
        pltpu.sync_copy(buf_b, out.at[pl.ds(base + _CHUNK, _CHUNK)])

    @functools.partial(
        pl.kernel,
        mesh=_mesh(),
        out_type=jax.ShapeDtypeStruct((2, _B, _D), jnp.float32),
        scratch_types=[
            pltpu.VMEM((bpw,), jnp.int32),
            pltpu.VMEM((_CHUNK, _D), jnp.float32),
            pltpu.VMEM((_CHUNK, _D), jnp.float32),
            pltpu.VMEM((_CHUNK, _D), jnp.float32),
            pltpu.SemaphoreType.DMA,
            pltpu.SemaphoreType.DMA,
        ],
        compiler_params=pltpu.CompilerParams(use_tc_tiling_on_sc=True),
    )
    def _gather_item(tbl, idx, ux, out, idx_v, buf_a, buf_b, pbuf,
                     sem_a, sem_b):
        base = _worker_id() * bpw
        pltpu.sync_copy(idx.at[pl.ds(base, bpw)], idx_v)
        _enqueue_rows(tbl, idx_v, buf_a, sem_a, 0)
        _enqueue_rows(tbl, idx_v, buf_b, sem_b, 1)
        for c in range(nchunk):
            pltpu.sync_copy(ux.at[pl.ds(base + c * _CHUNK, _CHUNK)], pbuf)
            pltpu.sync_copy(pbuf, out.at[0, pl.ds(base + c * _CHUNK, _CHUNK)])
        pltpu.make_async_copy(tbl.at[pl.ds(0, _CHUNK)], buf_a, sem_a).wait()
        pltpu.sync_copy(buf_a, out.at[1, pl.ds(base, _CHUNK)])
        pltpu.make_async_copy(tbl.at[pl.ds(0, _CHUNK)], buf_b, sem_b).wait()
        pltpu.sync_copy(buf_b, out.at[1, pl.ds(base + _CHUNK, _CHUNK)])

    user_x = _gather_user(user_table, user_idx.astype(jnp.int32))
    return _gather_item(item_table, item_idx.astype(jnp.int32), user_x)


# speedup vs baseline: 1.0225x; 1.0225x over previous
import functools

import jax
import jax.numpy as jnp
from jax import lax
from jax.experimental import pallas as pl
from jax.experimental.pallas import tpu as pltpu
from jax.experimental.pallas import tpu_sc as plsc

_B = 16384
_D = 64
_CHUNK = 256


def kernel(user_table, item_table, user_idx, item_idx):
    info = plsc.get_sparse_core_info()
    nw = info.num_cores * info.num_subcores
    bpw = _B // nw

    mesh = plsc.VectorSubcoreMesh(core_axis_name="c", subcore_axis_name="s")

    @functools.partial(
        pl.kernel,
        mesh=mesh,
        out_type=jax.ShapeDtypeStruct((2, _B, _D), jnp.float32),
        scratch_types=[
            pltpu.VMEM((bpw,), jnp.int32),
            pltpu.VMEM((bpw,), jnp.int32),
            pltpu.VMEM((_CHUNK, _D), jnp.float32),
            pltpu.VMEM((_CHUNK, _D), jnp.float32),
            pltpu.SemaphoreType.DMA,
            pltpu.SemaphoreType.DMA,
        ],
        compiler_params=pltpu.CompilerParams(use_tc_tiling_on_sc=True),
    )
    def _emb(ut, it, ui, ii, out, uidx_s, iidx_s, urows_v, irows_v,
             usem, isem):
        wid = lax.axis_index("s") * info.num_cores + lax.axis_index("c")
        base = wid * bpw
        pltpu.sync_copy(ui.at[pl.ds(base, bpw)], uidx_s)
        pltpu.sync_copy(ii.at[pl.ds(base, bpw)], iidx_s)

        def enqueue(tbl, idx_s, rows_v, sem, c):
            def body(g, carry):
                vec = idx_s[pl.ds(c * _CHUNK + g * 16, 16)]
                for k in range(16):
                    pltpu.async_copy(tbl.at[vec[k]], rows_v.at[g * 16 + k],
                                     sem)
                return carry
            lax.fori_loop(0, _CHUNK // 16, body, 0)

        nchunk = bpw // _CHUNK
        enqueue(ut, uidx_s, urows_v, usem, 0)
        enqueue(it, iidx_s, irows_v, isem, 0)
        for c in range(nchunk):
            pltpu.make_async_copy(ut.at[pl.ds(0, _CHUNK)], urows_v, usem).wait()
            pltpu.sync_copy(urows_v,
                            out.at[0, pl.ds(base + c * _CHUNK, _CHUNK)])
            if c + 1 < nchunk:
                enqueue(ut, uidx_s, urows_v, usem, c + 1)
            pltpu.make_async_copy(it.at[pl.ds(0, _CHUNK)], irows_v, isem).wait()
            pltpu.sync_copy(irows_v,
                            out.at[1, pl.ds(base + c * _CHUNK, _CHUNK)])
            if c + 1 < nchunk:
                enqueue(it, iidx_s, irows_v, isem, c + 1)

    return _emb(user_table, item_table,
                user_idx.astype(jnp.int32), item_idx.astype(jnp.int32))
